# Initial kernel scaffold; baseline (speedup 1.0000x reference)
import jax
import jax.numpy as jnp
from jax.experimental import pallas as pl


def kernel(inputs, W, b):
    raise NotImplementedError("write your pallas kernel here")


# speedup vs baseline: 3.2252x; 3.2252x over previous
import jax
import jax.numpy as jnp
from jax import lax
from jax.experimental import pallas as pl
from jax.experimental.pallas import tpu as pltpu

_TOP_K = 2
_E = 64
_D = 2048
_ALPHA = 0.01


def _router_block(i, n_blocks, n_tok, x_ref, w_ref, b_ref,
                  mask_ref, wout_ref, loss_ref,
                  acc_mask, acc_prob, acc_z):
    l = lax.dot_general(
        x_ref[...], w_ref[...],
        dimension_numbers=(((1,), (1,)), ((), ())),
        preferred_element_type=jnp.float32,
    ) + b_ref[...]

    m = jnp.max(l, axis=-1, keepdims=True)
    e = jnp.exp(l - m)
    s = jnp.sum(e, axis=-1, keepdims=True)
    p = e / s
    lse = m + jnp.log(s)

    iota = lax.broadcasted_iota(jnp.int32, p.shape, 1)
    m1 = jnp.max(p, axis=-1, keepdims=True)
    i1 = jnp.min(jnp.where(p == m1, iota, _E), axis=-1, keepdims=True)
    sel1 = iota == i1
    p2 = jnp.where(sel1, -jnp.inf, p)
    m2 = jnp.max(p2, axis=-1, keepdims=True)
    i2 = jnp.min(jnp.where(p2 == m2, iota, _E), axis=-1, keepdims=True)
    mask = jnp.where(sel1 | (iota == i2), 1.0, 0.0).astype(jnp.float32)

    mask_ref[...] = mask
    wout_ref[...] = p * mask

    @pl.when(i == 0)
    def _init():
        acc_mask[...] = jnp.zeros_like(acc_mask)
        acc_prob[...] = jnp.zeros_like(acc_prob)
        acc_z[...] = jnp.zeros_like(acc_z)

    acc_mask[...] += jnp.sum(mask, axis=0, keepdims=True)
    acc_prob[...] += jnp.sum(p, axis=0, keepdims=True)
    acc_z[...] += jnp.sum(lse * lse).reshape(1, 1)

    @pl.when(i == n_blocks - 1)
    def _final():
        inv_n = 1.0 / n_tok
        aux = _ALPHA * _E * jnp.sum(
            (acc_mask[...] * inv_n) * (acc_prob[...] * inv_n))
        loss_ref[...] = aux.reshape(1, 1) + acc_z[...] * inv_n


def kernel(inputs, W, b):
    orig_dtype = inputs.dtype
    x = inputs.astype(jnp.float32).reshape(-1, _D)
    n_tok = x.shape[0]
    T = 512
    n_blocks = n_tok // T
    b2 = b.reshape(1, _E).astype(jnp.float32)

    def body(x_ref, w_ref, b_ref, mask_ref, wout_ref, loss_ref,
             acc_mask, acc_prob, acc_z):
        i = pl.program_id(0)
        _router_block(i, n_blocks, n_tok, x_ref, w_ref, b_ref,
                      mask_ref, wout_ref, loss_ref, acc_mask, acc_prob, acc_z)

    mask, wout, loss = pl.pallas_call(
        body,
        grid=(n_blocks,),
        in_specs=[
            pl.BlockSpec((T, _D), lambda i: (i, 0)),
            pl.BlockSpec((_E, _D), lambda i: (0, 0)),
            pl.BlockSpec((1, _E), lambda i: (0, 0)),
        ],
        out_specs=[
            pl.BlockSpec((T, _E), lambda i: (i, 0)),
            pl.BlockSpec((T, _E), lambda i: (i, 0)),
            pl.BlockSpec((1, 1), lambda i: (0, 0)),
        ],
        out_shape=[
            jax.ShapeDtypeStruct((n_tok, _E), jnp.float32),
            jax.ShapeDtypeStruct((n_tok, _E), jnp.float32),
            jax.ShapeDtypeStruct((1, 1), jnp.float32),
        ],
        scratch_shapes=[
            pltpu.VMEM((1, _E), jnp.float32),
            pltpu.VMEM((1, _E), jnp.float32),
            pltpu.VMEM((1, 1), jnp.float32),
        ],
    )(x, W, b2)

    return (mask, wout.astype(orig_dtype), loss[0, 0])


# speedup vs baseline: 3.5628x; 1.1047x over previous
import jax
import jax.numpy as jnp
from jax import lax
from jax.experimental import pallas as pl
from jax.experimental.pallas import tpu as pltpu

_TOP_K = 2
_E = 64
_D = 2048
_ALPHA = 0.01


def _router_block(i, n_blocks, n_tok, x_ref, w_ref, b_ref,
                  mask_ref, wout_ref, loss_ref,
                  acc_mask, acc_prob, acc_z):
    l = lax.dot_general(
        x_ref[...], w_ref[...],
        dimension_numbers=(((1,), (1,)), ((), ())),
        preferred_element_type=jnp.float32,
    ) + b_ref[...]

    m = jnp.max(l, axis=-1, keepdims=True)
    e = jnp.exp(l - m)
    s = jnp.sum(e, axis=-1, keepdims=True)
    p = e * (1.0 / s)
    lse = m + jnp.log(s)

    sel1 = e >= 1.0
    m2 = jnp.max(jnp.where(sel1, -1.0, e), axis=-1, keepdims=True)
    mask = jnp.where(sel1 | (e >= m2), 1.0, 0.0).astype(jnp.float32)

    mask_ref[...] = mask
    wout_ref[...] = p * mask

    @pl.when(i == 0)
    def _init():
        acc_mask[...] = jnp.zeros_like(acc_mask)
        acc_prob[...] = jnp.zeros_like(acc_prob)
        acc_z[...] = jnp.zeros_like(acc_z)

    acc_mask[...] += jnp.sum(mask, axis=0, keepdims=True)
    acc_prob[...] += jnp.sum(p, axis=0, keepdims=True)
    acc_z[...] += jnp.sum(lse * lse).reshape(1, 1)

    @pl.when(i == n_blocks - 1)
    def _final():
        inv_n = 1.0 / n_tok
        aux = _ALPHA * _E * jnp.sum(
            (acc_mask[...] * inv_n) * (acc_prob[...] * inv_n))
        loss_ref[...] = aux.reshape(1, 1) + acc_z[...] * inv_n


def kernel(inputs, W, b):
    orig_dtype = inputs.dtype
    x = inputs.astype(jnp.float32).reshape(-1, _D)
    n_tok = x.shape[0]
    T = 512
    n_blocks = n_tok // T
    b2 = b.reshape(1, _E).astype(jnp.float32)

    def body(x_ref, w_ref, b_ref, mask_ref, wout_ref, loss_ref,
             acc_mask, acc_prob, acc_z):
        i = pl.program_id(0)
        _router_block(i, n_blocks, n_tok, x_ref, w_ref, b_ref,
                      mask_ref, wout_ref, loss_ref, acc_mask, acc_prob, acc_z)

    mask, wout, loss = pl.pallas_call(
        body,
        grid=(n_blocks,),
        in_specs=[
            pl.BlockSpec((T, _D), lambda i: (i, 0)),
            pl.BlockSpec((_E, _D), lambda i: (0, 0)),
            pl.BlockSpec((1, _E), lambda i: (0, 0)),
        ],
        out_specs=[
            pl.BlockSpec((T, _E), lambda i: (i, 0)),
            pl.BlockSpec((T, _E), lambda i: (i, 0)),
            pl.BlockSpec((1, 1), lambda i: (0, 0)),
        ],
        out_shape=[
            jax.ShapeDtypeStruct((n_tok, _E), jnp.float32),
            jax.ShapeDtypeStruct((n_tok, _E), jnp.float32),
            jax.ShapeDtypeStruct((1, 1), jnp.float32),
        ],
        scratch_shapes=[
            pltpu.VMEM((1, _E), jnp.float32),
            pltpu.VMEM((1, _E), jnp.float32),
            pltpu.VMEM((1, 1), jnp.float32),
        ],
    )(x, W, b2)

    return (mask, wout.astype(orig_dtype), loss[0, 0])


# speedup vs baseline: 4.1141x; 1.1547x over previous
import jax
import jax.numpy as jnp
from jax import lax
from jax.experimental import pallas as pl
from jax.experimental.pallas import tpu as pltpu

_TOP_K = 2
_E = 64
_D = 2048
_ALPHA = 0.01


def _router_block(i, n_blocks, n_tok, x_ref, w_ref, b_ref,
                  mask_ref, wout_ref, loss_ref,
                  acc_mask, acc_prob, acc_z):
    l = lax.dot_general(
        x_ref[...], w_ref[...],
        dimension_numbers=(((1,), (1,)), ((), ())),
        preferred_element_type=jnp.float32,
    ) + b_ref[...]

    m = jnp.max(l, axis=-1, keepdims=True)
    e = jnp.exp(l - m)
    s = jnp.sum(e, axis=-1, keepdims=True)
    p = e * (1.0 / s)
    lse = m + jnp.log(s)

    sel1 = e >= 1.0
    m2 = jnp.max(jnp.where(sel1, -1.0, e), axis=-1, keepdims=True)
    mask = jnp.where(sel1 | (e >= m2), 1.0, 0.0).astype(jnp.float32)

    mask_ref[...] = mask
    wout_ref[...] = p * mask

    @pl.when(i == 0)
    def _init():
        acc_mask[...] = jnp.zeros_like(acc_mask)
        acc_prob[...] = jnp.zeros_like(acc_prob)
        acc_z[...] = jnp.zeros_like(acc_z)

    acc_mask[...] += jnp.sum(mask, axis=0, keepdims=True)
    acc_prob[...] += jnp.sum(p, axis=0, keepdims=True)
    acc_z[...] += jnp.sum(lse * lse).reshape(1, 1)

    @pl.when(i == n_blocks - 1)
    def _final():
        inv_n = 1.0 / n_tok
        aux = _ALPHA * _E * jnp.sum(
            (acc_mask[...] * inv_n) * (acc_prob[...] * inv_n))
        loss_ref[...] = aux.reshape(1, 1) + acc_z[...] * inv_n


def kernel(inputs, W, b):
    orig_dtype = inputs.dtype
    x = inputs.astype(jnp.float32).reshape(-1, _D)
    n_tok = x.shape[0]
    T = 1024
    n_blocks = n_tok // T
    b2 = b.reshape(1, _E).astype(jnp.float32)

    def body(x_ref, w_ref, b_ref, mask_ref, wout_ref, loss_ref,
             acc_mask, acc_prob, acc_z):
        i = pl.program_id(0)
        _router_block(i, n_blocks, n_tok, x_ref, w_ref, b_ref,
                      mask_ref, wout_ref, loss_ref, acc_mask, acc_prob, acc_z)

    mask, wout, loss = pl.pallas_call(
        body,
        grid=(n_blocks,),
        in_specs=[
            pl.BlockSpec((T, _D), lambda i: (i, 0)),
            pl.BlockSpec((_E, _D), lambda i: (0, 0)),
            pl.BlockSpec((1, _E), lambda i: (0, 0)),
        ],
        out_specs=[
            pl.BlockSpec((T, _E), lambda i: (i, 0)),
            pl.BlockSpec((T, _E), lambda i: (i, 0)),
            pl.BlockSpec((1, 1), lambda i: (0, 0)),
        ],
        out_shape=[
            jax.ShapeDtypeStruct((n_tok, _E), jnp.float32),
            jax.ShapeDtypeStruct((n_tok, _E), jnp.float32),
            jax.ShapeDtypeStruct((1, 1), jnp.float32),
        ],
        scratch_shapes=[
            pltpu.VMEM((1, _E), jnp.float32),
            pltpu.VMEM((1, _E), jnp.float32),
            pltpu.VMEM((1, 1), jnp.float32),
        ],
    )(x, W, b2)

    return (mask, wout.astype(orig_dtype), loss[0, 0])


# speedup vs baseline: 4.2404x; 1.0307x over previous
import jax
import jax.numpy as jnp
from jax import lax
from jax.experimental import pallas as pl
from jax.experimental.pallas import tpu as pltpu

_TOP_K = 2
_E = 64
_D = 2048
_ALPHA = 0.01


def _router_block(i, n_blocks, n_tok, x_ref, w_ref, b_ref,
                  mask_ref, wout_ref, loss_ref,
                  acc_mask, acc_prob, acc_z):
    l = lax.dot_general(
        x_ref[...], w_ref[...],
        dimension_numbers=(((1,), (1,)), ((), ())),
        preferred_element_type=jnp.float32,
    ) + b_ref[...]

    m = jnp.max(l, axis=-1, keepdims=True)
    e = jnp.exp(l - m)
    s = jnp.sum(e, axis=-1, keepdims=True)
    p = e * (1.0 / s)
    lse = m + jnp.log(s)

    sel1 = e >= 1.0
    m2 = jnp.max(jnp.where(sel1, -1.0, e), axis=-1, keepdims=True)
    mask = jnp.where(sel1 | (e >= m2), 1.0, 0.0).astype(jnp.float32)

    mask_ref[...] = mask
    wout_ref[...] = p * mask

    @pl.when(i == 0)
    def _init():
        acc_mask[...] = jnp.zeros_like(acc_mask)
        acc_prob[...] = jnp.zeros_like(acc_prob)
        acc_z[...] = jnp.zeros_like(acc_z)

    acc_mask[...] += jnp.sum(mask, axis=0, keepdims=True)
    acc_prob[...] += jnp.sum(p, axis=0, keepdims=True)
    acc_z[...] += jnp.sum(lse * lse).reshape(1, 1)

    @pl.when(i == n_blocks - 1)
    def _final():
        inv_n = 1.0 / n_tok
        aux = _ALPHA * _E * jnp.sum(
            (acc_mask[...] * inv_n) * (acc_prob[...] * inv_n))
        loss_ref[...] = aux.reshape(1, 1) + acc_z[...] * inv_n


def kernel(inputs, W, b):
    orig_dtype = inputs.dtype
    x = inputs.astype(jnp.float32).reshape(-1, _D)
    n_tok = x.shape[0]
    T = 2048
    n_blocks = n_tok // T
    b2 = b.reshape(1, _E).astype(jnp.float32)

    def body(x_ref, w_ref, b_ref, mask_ref, wout_ref, loss_ref,
             acc_mask, acc_prob, acc_z):
        i = pl.program_id(0)
        _router_block(i, n_blocks, n_tok, x_ref, w_ref, b_ref,
                      mask_ref, wout_ref, loss_ref, acc_mask, acc_prob, acc_z)

    mask, wout, loss = pl.pallas_call(
        body,
        grid=(n_blocks,),
        in_specs=[
            pl.BlockSpec((T, _D), lambda i: (i, 0)),
            pl.BlockSpec((_E, _D), lambda i: (0, 0)),
            pl.BlockSpec((1, _E), lambda i: (0, 0)),
        ],
        out_specs=[
            pl.BlockSpec((T, _E), lambda i: (i, 0)),
            pl.BlockSpec((T, _E), lambda i: (i, 0)),
            pl.BlockSpec((1, 1), lambda i: (0, 0)),
        ],
        out_shape=[
            jax.ShapeDtypeStruct((n_tok, _E), jnp.float32),
            jax.ShapeDtypeStruct((n_tok, _E), jnp.float32),
            jax.ShapeDtypeStruct((1, 1), jnp.float32),
        ],
        scratch_shapes=[
            pltpu.VMEM((1, _E), jnp.float32),
            pltpu.VMEM((1, _E), jnp.float32),
            pltpu.VMEM((1, 1), jnp.float32),
        ],
    )(x, W, b2)

    return (mask, wout.astype(orig_dtype), loss[0, 0])


# speedup vs baseline: 4.2544x; 1.0033x over previous
import jax
import jax.numpy as jnp
from jax import lax
from jax.experimental import pallas as pl
from jax.experimental.pallas import tpu as pltpu

_TOP_K = 2
_E = 64
_D = 2048
_ALPHA = 0.01


def _router_block(i, n_blocks, n_tok, x_ref, w_ref, b_ref,
                  mask_ref, wout_ref, loss_ref,
                  acc_mask, acc_prob, acc_z):
    l = lax.dot_general(
        x_ref[...], w_ref[...],
        dimension_numbers=(((1,), (1,)), ((), ())),
        preferred_element_type=jnp.float32,
    ) + b_ref[...]

    m = jnp.max(l, axis=-1, keepdims=True)
    e = jnp.exp(l - m)
    s = jnp.sum(e, axis=-1, keepdims=True)
    p = e * (1.0 / s)
    lse = m + jnp.log(s)

    sel1 = e >= 1.0
    m2 = jnp.max(jnp.where(sel1, -1.0, e), axis=-1, keepdims=True)
    mask = jnp.where(sel1 | (e >= m2), 1.0, 0.0).astype(jnp.float32)

    mask_ref[...] = mask
    wout_ref[...] = p * mask

    @pl.when(i == 0)
    def _init():
        acc_mask[...] = jnp.zeros_like(acc_mask)
        acc_prob[...] = jnp.zeros_like(acc_prob)
        acc_z[...] = jnp.zeros_like(acc_z)

    acc_mask[...] += jnp.sum(mask, axis=0, keepdims=True)
    acc_prob[...] += jnp.sum(p, axis=0, keepdims=True)
    acc_z[...] += jnp.sum(lse * lse).reshape(1, 1)

    @pl.when(i == n_blocks - 1)
    def _final():
        inv_n = 1.0 / n_tok
        aux = _ALPHA * _E * jnp.sum(
            (acc_mask[...] * inv_n) * (acc_prob[...] * inv_n))
        loss_ref[...] = aux.reshape(1, 1) + acc_z[...] * inv_n


def kernel(inputs, W, b):
    orig_dtype = inputs.dtype
    x = inputs.astype(jnp.float32).reshape(-1, _D)
    n_tok = x.shape[0]
    T = 2048
    n_blocks = n_tok // T
    b2 = b.reshape(1, _E).astype(jnp.float32)

    def body(x_ref, w_ref, b_ref, mask_ref, wout_ref, loss_ref,
             acc_mask, acc_prob, acc_z):
        i = pl.program_id(0)
        _router_block(i, n_blocks, n_tok, x_ref, w_ref, b_ref,
                      mask_ref, wout_ref, loss_ref, acc_mask, acc_prob, acc_z)

    mask, wout, loss = pl.pallas_call(
        body,
        grid=(n_blocks,),
        in_specs=[
            pl.BlockSpec((T, _D), lambda i: (i, 0)),
            pl.BlockSpec((_E, _D), lambda i: (0, 0)),
            pl.BlockSpec((1, _E), lambda i: (0, 0)),
        ],
        out_specs=[
            pl.BlockSpec((T, _E), lambda i: (i, 0)),
            pl.BlockSpec((T, _E), lambda i: (i, 0)),
            pl.BlockSpec((1, 1), lambda i: (0, 0)),
        ],
        out_shape=[
            jax.ShapeDtypeStruct((n_tok, _E), jnp.float32),
            jax.ShapeDtypeStruct((n_tok, _E), jnp.float32),
            jax.ShapeDtypeStruct((1, 1), jnp.float32),
        ],
        scratch_shapes=[
            pltpu.VMEM((1, _E), jnp.float32),
            pltpu.VMEM((1, _E), jnp.float32),
            pltpu.VMEM((1, 1), jnp.float32),
        ],
        compiler_params=pltpu.CompilerParams(
            vmem_limit_bytes=100 * 1024 * 1024,
        ),
    )(x, W, b2)

    return (mask, wout.astype(orig_dtype), loss[0, 0])
